# baseline (speedup 1.0000x reference)
import jax
import jax.numpy as jnp
from jax.experimental import pallas as pl
from jax.experimental.pallas import tpu as pltpu

_FANOUT = 16
_BLK = 1024


def _fused_body(e_ref, w0_ref, b0_ref, w1_ref, b1_ref, s_ref, out_ref):
    e = e_ref[...].astype(jnp.bfloat16)
    t1 = jax.nn.sigmoid(
        jnp.dot(e, w1_ref[...], preferred_element_type=jnp.float32) + b1_ref[...]
    )
    p0 = jax.nn.sigmoid(
        jnp.dot(e, w0_ref[...], preferred_element_type=jnp.float32) + b0_ref[...]
    )
    p0_exp = jnp.dot(
        p0.astype(jnp.bfloat16), s_ref[...], preferred_element_type=jnp.float32
    )
    out_ref[...] = p0_exp * t1


def kernel(patient_embedding, y_true0, y_true1, W0, b0, W1, b1):
    B, D = patient_embedding.shape
    DIM0 = W0.shape[1]
    DIM1 = W1.shape[1]
    S = jnp.kron(
        jnp.eye(DIM0, dtype=jnp.bfloat16), jnp.ones((1, _FANOUT), jnp.bfloat16)
    )
    W0b = W0.astype(jnp.bfloat16)
    W1b = W1.astype(jnp.bfloat16)
    b0r = b0.reshape(1, DIM0)
    b1r = b1.reshape(1, DIM1)
    return pl.pallas_call(
        _fused_body,
        grid=(B // _BLK,),
        in_specs=[
            pl.BlockSpec((_BLK, D), lambda i: (i, 0)),
            pl.BlockSpec((D, DIM0), lambda i: (0, 0)),
            pl.BlockSpec((1, DIM0), lambda i: (0, 0)),
            pl.BlockSpec((D, DIM1), lambda i: (0, 0)),
            pl.BlockSpec((1, DIM1), lambda i: (0, 0)),
            pl.BlockSpec((DIM0, DIM1), lambda i: (0, 0)),
        ],
        out_specs=pl.BlockSpec((_BLK, DIM1), lambda i: (i, 0)),
        out_shape=jax.ShapeDtypeStruct((B, DIM1), jnp.float32),
        compiler_params=pltpu.CompilerParams(dimension_semantics=("parallel",)),
    )(patient_embedding, W0b, b0r, W1b, b1r, S)


# speedup vs baseline: 1.1391x; 1.1391x over previous
import jax
import jax.numpy as jnp
from jax.experimental import pallas as pl
from jax.experimental.pallas import tpu as pltpu

_FANOUT = 16
_BLK = 1024


def _sigmoid(x):
    return 0.5 * jnp.tanh(0.5 * x) + 0.5


def _fused_body(e_ref, w0_ref, b0_ref, w1_ref, b1_ref, s_ref, out_ref):
    e = e_ref[...]
    t1 = _sigmoid(
        jnp.dot(e, w1_ref[...], preferred_element_type=jnp.float32) + b1_ref[...]
    )
    p0 = _sigmoid(
        jnp.dot(e, w0_ref[...], preferred_element_type=jnp.float32) + b0_ref[...]
    )
    p0_exp = jnp.dot(p0, s_ref[...], preferred_element_type=jnp.float32)
    out_ref[...] = p0_exp * t1


def kernel(patient_embedding, y_true0, y_true1, W0, b0, W1, b1):
    B, D = patient_embedding.shape
    DIM0 = W0.shape[1]
    DIM1 = W1.shape[1]
    S = jnp.kron(jnp.eye(DIM0, dtype=jnp.float32), jnp.ones((1, _FANOUT), jnp.float32))
    b0r = b0.reshape(1, DIM0)
    b1r = b1.reshape(1, DIM1)
    return pl.pallas_call(
        _fused_body,
        grid=(B // _BLK,),
        in_specs=[
            pl.BlockSpec((_BLK, D), lambda i: (i, 0)),
            pl.BlockSpec((D, DIM0), lambda i: (0, 0)),
            pl.BlockSpec((1, DIM0), lambda i: (0, 0)),
            pl.BlockSpec((D, DIM1), lambda i: (0, 0)),
            pl.BlockSpec((1, DIM1), lambda i: (0, 0)),
            pl.BlockSpec((DIM0, DIM1), lambda i: (0, 0)),
        ],
        out_specs=pl.BlockSpec((_BLK, DIM1), lambda i: (i, 0)),
        out_shape=jax.ShapeDtypeStruct((B, DIM1), jnp.float32),
        compiler_params=pltpu.CompilerParams(dimension_semantics=("parallel",)),
    )(patient_embedding, W0, b0r, W1, b1r, S)


# speedup vs baseline: 1.3588x; 1.1929x over previous
import jax
import jax.numpy as jnp
from jax.experimental import pallas as pl
from jax.experimental.pallas import tpu as pltpu

_FANOUT = 16
_BLK = 2048


def _sigmoid(x):
    return 0.5 * jnp.tanh(0.5 * x) + 0.5


def _fused_body(e_ref, w0_ref, b0_ref, w1_ref, b1_ref, s_ref, out_ref):
    e = e_ref[...]
    t1 = _sigmoid(
        jnp.dot(e, w1_ref[...], preferred_element_type=jnp.float32) + b1_ref[...]
    )
    p0 = _sigmoid(
        jnp.dot(e, w0_ref[...], preferred_element_type=jnp.float32) + b0_ref[...]
    )
    p0_exp = jnp.dot(p0, s_ref[...], preferred_element_type=jnp.float32)
    out_ref[...] = p0_exp * t1


def kernel(patient_embedding, y_true0, y_true1, W0, b0, W1, b1):
    B, D = patient_embedding.shape
    DIM0 = W0.shape[1]
    DIM1 = W1.shape[1]
    S = jnp.kron(jnp.eye(DIM0, dtype=jnp.float32), jnp.ones((1, _FANOUT), jnp.float32))
    b0r = b0.reshape(1, DIM0)
    b1r = b1.reshape(1, DIM1)
    return pl.pallas_call(
        _fused_body,
        grid=(B // _BLK,),
        in_specs=[
            pl.BlockSpec((_BLK, D), lambda i: (i, 0)),
            pl.BlockSpec((D, DIM0), lambda i: (0, 0)),
            pl.BlockSpec((1, DIM0), lambda i: (0, 0)),
            pl.BlockSpec((D, DIM1), lambda i: (0, 0)),
            pl.BlockSpec((1, DIM1), lambda i: (0, 0)),
            pl.BlockSpec((DIM0, DIM1), lambda i: (0, 0)),
        ],
        out_specs=pl.BlockSpec((_BLK, DIM1), lambda i: (i, 0)),
        out_shape=jax.ShapeDtypeStruct((B, DIM1), jnp.float32),
        compiler_params=pltpu.CompilerParams(dimension_semantics=("parallel",)),
    )(patient_embedding, W0, b0r, W1, b1r, S)


# speedup vs baseline: 1.4069x; 1.0354x over previous
import jax
import jax.numpy as jnp
from jax.experimental import pallas as pl
from jax.experimental.pallas import tpu as pltpu

_FANOUT = 16
_BLK = 4096


def _sigmoid(x):
    return 0.5 * jnp.tanh(0.5 * x) + 0.5


def _fused_body(e_ref, w0_ref, b0_ref, w1_ref, b1_ref, s_ref, out_ref):
    e = e_ref[...]
    t1 = _sigmoid(
        jnp.dot(e, w1_ref[...], preferred_element_type=jnp.float32) + b1_ref[...]
    )
    p0 = _sigmoid(
        jnp.dot(e, w0_ref[...], preferred_element_type=jnp.float32) + b0_ref[...]
    )
    p0_exp = jnp.dot(p0, s_ref[...], preferred_element_type=jnp.float32)
    out_ref[...] = p0_exp * t1


def kernel(patient_embedding, y_true0, y_true1, W0, b0, W1, b1):
    B, D = patient_embedding.shape
    DIM0 = W0.shape[1]
    DIM1 = W1.shape[1]
    S = jnp.kron(jnp.eye(DIM0, dtype=jnp.float32), jnp.ones((1, _FANOUT), jnp.float32))
    b0r = b0.reshape(1, DIM0)
    b1r = b1.reshape(1, DIM1)
    return pl.pallas_call(
        _fused_body,
        grid=(B // _BLK,),
        in_specs=[
            pl.BlockSpec((_BLK, D), lambda i: (i, 0)),
            pl.BlockSpec((D, DIM0), lambda i: (0, 0)),
            pl.BlockSpec((1, DIM0), lambda i: (0, 0)),
            pl.BlockSpec((D, DIM1), lambda i: (0, 0)),
            pl.BlockSpec((1, DIM1), lambda i: (0, 0)),
            pl.BlockSpec((DIM0, DIM1), lambda i: (0, 0)),
        ],
        out_specs=pl.BlockSpec((_BLK, DIM1), lambda i: (i, 0)),
        out_shape=jax.ShapeDtypeStruct((B, DIM1), jnp.float32),
        compiler_params=pltpu.CompilerParams(dimension_semantics=("parallel",)),
    )(patient_embedding, W0, b0r, W1, b1r, S)
